# baseline (speedup 1.0000x reference)
import functools

import jax
import jax.numpy as jnp
from jax import lax
from jax.experimental import pallas as pl
from jax.experimental.pallas import tpu as pltpu
from jax.experimental.pallas import tpu_sc as plsc

B = 16384
N_NUM = 13
N_CAT = 26
N_TOK = N_NUM + N_CAT
CARD = 100000
D = 32

NC = 2
NS = 16
NW = NC * NS
S_PER_W = B // NW

_mesh = plsc.VectorSubcoreMesh(core_axis_name="c", subcore_axis_name="s")


@functools.partial(
    pl.kernel,
    out_type=jax.ShapeDtypeStruct((N_CAT * B, D), jnp.float32),
    mesh=_mesh,
    compiler_params=pltpu.CompilerParams(use_tc_tiling_on_sc=False),
    scratch_types=[
        pltpu.VMEM((S_PER_W,), jnp.int32),
        pltpu.VMEM((4, 128), jnp.int32),
        pltpu.VMEM((S_PER_W, D), jnp.float32),
        pltpu.SemaphoreType.DMA,
    ],
)
def _cat_gather(xcat_hbm, table_hbm, out_hbm, xc_v, idx_v, rows_v, sem):
    wid = lax.axis_index("s") * NC + lax.axis_index("c")
    b0 = wid * S_PER_W

    def per_field(j, carry):
        pltpu.sync_copy(xcat_hbm.at[pl.ds(j * B + b0, S_PER_W)], xc_v)
        off = j * CARD

        def grp(g, c2):
            xv = xc_v[pl.ds(g * 16, 16)]
            v = jnp.maximum(xv, 0) + off
            zero = jnp.zeros((16,), jnp.int32)
            step = 4 * SPLIT - 1
            q = (jnp.where(v >= SPLIT, step, zero)
                 + jnp.where(v >= 2 * SPLIT, step, zero)
                 + jnp.where(v >= 3 * SPLIT, step, zero))
            idx_v[g // 8, pl.ds((g % 8) * 16, 16)] = v * 4 - q
            return c2

        lax.fori_loop(0, S_PER_W // 16, grp, None)

        gathers = [
            pltpu.async_copy(table_hbm.at[idx_v.at[q]],
                             rows_v.at[pl.ds(q * 128, 128)], sem)
            for q in range(S_PER_W // 128)
        ]
        for cp in gathers:
            cp.wait()
        pltpu.sync_copy(rows_v, out_hbm.at[pl.ds(j * B + b0, S_PER_W)])
        return carry

    lax.fori_loop(0, N_CAT, per_field, None)


TROWS = N_CAT * CARD
SPLIT = 650112
CH = 384
NBLK = SPLIT // CH


def _conv_body(t0, t1, t2, t3, o_ref):
    for q, t in enumerate((t0, t1, t2, t3)):
        o_ref[:, q * D:(q + 1) * D] = jnp.transpose(t[...], (1, 0))


def _make_spec(q):
    return pl.BlockSpec((D, CH), lambda i, q=q: (0, q * NBLK + i))


def _tc_relayout_table(tableT):
    return pl.pallas_call(
        _conv_body,
        grid=(NBLK,),
        in_specs=[_make_spec(0), _make_spec(1), _make_spec(2), _make_spec(3)],
        out_specs=pl.BlockSpec((CH, 4 * D), lambda i: (i, 0)),
        out_shape=jax.ShapeDtypeStruct((SPLIT, 4 * D), jnp.float32),
    )(tableT, tableT, tableT, tableT)


BL = 512


def _asm_body(x_ref, w_ref, b_ref, c_ref, o_ref):
    o_ref[0:N_NUM] = (x_ref[...][:, None, :] * w_ref[...][:, :, None]
                      + b_ref[...][:, :, None])
    o_ref[N_NUM:N_TOK] = jnp.transpose(c_ref[...], (0, 2, 1))


def _tc_assemble(xnT, num_weight, num_bias, catc3):
    return pl.pallas_call(
        _asm_body,
        grid=(B // BL,),
        in_specs=[
            pl.BlockSpec((N_NUM, BL), lambda i: (0, i)),
            pl.BlockSpec((N_NUM, D), lambda i: (0, 0)),
            pl.BlockSpec((N_NUM, D), lambda i: (0, 0)),
            pl.BlockSpec((N_CAT, BL, D), lambda i: (0, i, 0)),
        ],
        out_specs=pl.BlockSpec((N_TOK, D, BL), lambda i: (0, 0, i)),
        out_shape=jax.ShapeDtypeStruct((N_TOK, D, B), jnp.float32),
    )(xnT, num_weight, num_bias, catc3)


def kernel(x_num, x_cat, num_weight, num_bias, cat_table):
    xcT = jnp.transpose(x_cat.astype(jnp.int32), (1, 0)).reshape(N_CAT * B)
    xnT = jnp.transpose(x_num, (1, 0))
    packed = _tc_relayout_table(jnp.transpose(cat_table, (1, 0)))
    catc = _cat_gather(xcT, packed.reshape(4 * SPLIT, D))
    catc3 = catc.reshape(N_CAT, B, D)
    out3 = _tc_assemble(xnT, num_weight, num_bias, catc3)
    return jnp.transpose(out3, (2, 0, 1))
